# baseline (speedup 1.0000x reference)
import functools

import jax
import jax.numpy as jnp
from jax.experimental import pallas as pl

CODES_DIM = 64
CODES_CNT = 8192
COMMIT = 0.25


_TB = 512
_CB = 256
_SCHUNK = 2048 // _CB


def _vq_dist_kernel(x_ref, cb_ref, val_ref, idx_ref):
    j = pl.program_id(1)
    x = x_ref[...]
    cbk = cb_ref[...]
    xy = jax.lax.dot_general(x.astype(jnp.bfloat16), cbk.astype(jnp.bfloat16),
                             (((1,), (1,)), ((), ())),
                             preferred_element_type=jnp.float32)
    x2 = jnp.sum(x * x, axis=1)[:, None]
    y2 = jnp.sum(cbk * cbk, axis=1)[None, :]
    d = (x2 + y2) - 2.0 * xy
    lmin = jnp.min(d, axis=1, keepdims=True)
    cols = jax.lax.broadcasted_iota(jnp.int32, (_TB, _CB), 1) + j * _CB
    larg = jnp.min(jnp.where(d <= lmin, cols, jnp.int32(2 ** 30)),
                   axis=1, keepdims=True)

    @pl.when(j == 0)
    def _():
        val_ref[...] = lmin
        idx_ref[...] = larg

    @pl.when((j > 0) & (j % _SCHUNK == 0))
    def _():
        val_ref[...] = val_ref[...].astype(jnp.bfloat16).astype(jnp.float32)

    @pl.when(j > 0)
    def _():
        prev_v = val_ref[...]
        prev_i = idx_ref[...]
        better = lmin < prev_v
        val_ref[...] = jnp.where(better, lmin, prev_v)
        idx_ref[...] = jnp.where(better, larg, prev_i)


def _vq_argmin(xt, cb):
    T = xt.shape[0]
    n_tb = T // _TB
    n_cb = CODES_CNT // _CB
    vals, idxs = pl.pallas_call(
        _vq_dist_kernel,
        grid=(n_tb, n_cb),
        in_specs=[
            pl.BlockSpec((_TB, CODES_DIM), lambda i, j: (i, 0)),
            pl.BlockSpec((_CB, CODES_DIM), lambda i, j: (j, 0)),
        ],
        out_specs=[
            pl.BlockSpec((_TB, 1), lambda i, j: (i, 0)),
            pl.BlockSpec((_TB, 1), lambda i, j: (i, 0)),
        ],
        out_shape=[
            jax.ShapeDtypeStruct((T, 1), jnp.float32),
            jax.ShapeDtypeStruct((T, 1), jnp.int32),
        ],
    )(xt, cb)
    return vals.reshape(T), idxs.reshape(T)


def _conv1d(x, w, s, p):
    return jax.lax.conv_general_dilated(
        x, w, (s,), [(p, p)], dimension_numbers=('NCH', 'OIH', 'NCH'))


def _convT1d(x, w, s, p):
    k = w.shape[2]
    w2 = jnp.transpose(jnp.flip(w, 2), (1, 0, 2))
    return jax.lax.conv_general_dilated(
        x, w2, (1,), [(k - 1 - p, k - 1 - p)], lhs_dilation=(s,),
        dimension_numbers=('NCH', 'OIH', 'NCH'))


def _bn(x):
    m = jnp.mean(x, axis=(0, 2), keepdims=True)
    v = jnp.var(x, axis=(0, 2), keepdims=True)
    return (x - m) / jnp.sqrt(v + 1e-5)


def _resblock(x, w1, w2):
    h = jax.nn.relu(x)
    h = _conv1d(h, w1, 1, 1)
    h = _bn(h)
    h = jax.nn.relu(h)
    h = _conv1d(h, w2, 1, 0)
    return _bn(x + h)


def kernel(x, cb, ew1, ew2, er1a, er1b, er2a, er2b, ew3, ew4,
           dw1, dw2, dr1a, dr1b, dr2a, dr2b, dw3, dw4):
    h = _conv1d(x, ew1, 2, 1)
    h = jax.nn.relu(h)
    h = _conv1d(h, ew2, 2, 1)
    h = _bn(h)
    h = _resblock(h, er1a, er1b)
    h = _resblock(h, er2a, er2b)
    h = jax.nn.relu(h)
    h = _conv1d(h, ew3, 2, 1)
    h = _bn(h)
    h = jax.nn.relu(h)
    z = _conv1d(h, ew4, 1, 1)

    b, c, L = z.shape
    xt = jnp.transpose(z, (1, 0, 2)).reshape(c, -1).T

    mind, ind = _vq_argmin(xt, cb)
    loss = (1.0 + COMMIT) * jnp.sum(mind) / (c * b * L)

    qs = jnp.take(cb, ind, axis=0)
    zq = jnp.transpose(qs.T.reshape(c, b, L), (1, 0, 2))

    h = _convT1d(zq, dw1, 1, 1)
    h = _bn(h)
    h = jax.nn.relu(h)
    h = _convT1d(h, dw2, 2, 1)
    h = _bn(h)
    h = _resblock(h, dr1a, dr1b)
    h = _resblock(h, dr2a, dr2b)
    h = jax.nn.relu(h)
    h = _convT1d(h, dw3, 2, 1)
    h = _bn(h)
    h = jax.nn.relu(h)
    h = _convT1d(h, dw4, 2, 1)
    x_rec = jnp.tanh(h)
    return (loss, x_rec)


# speedup vs baseline: 2.0481x; 2.0481x over previous
import functools

import jax
import jax.numpy as jnp
from jax.experimental import pallas as pl
from jax.experimental.pallas import tpu as pltpu
from jax.experimental.pallas import tpu_sc as plsc

CODES_DIM = 64
CODES_CNT = 8192
COMMIT = 0.25


_TB = 2048
_CB = 2048
_SCHUNK = 2048 // _CB


def _vq_dist_kernel(x_ref, cb_ref, val_ref, idx_ref):
    j = pl.program_id(1)
    x = x_ref[...]
    cbk = cb_ref[...]
    xy = jax.lax.dot_general(x.astype(jnp.bfloat16), cbk.astype(jnp.bfloat16),
                             (((1,), (1,)), ((), ())),
                             preferred_element_type=jnp.float32)
    x2 = jnp.sum(x * x, axis=1)[:, None]
    y2 = jnp.sum(cbk * cbk, axis=1)[None, :]
    d = (x2 + y2) - 2.0 * xy
    lmin = jnp.min(d, axis=1, keepdims=True)
    cols = jax.lax.broadcasted_iota(jnp.int32, (_TB, _CB), 1)
    lidx = jnp.min(jnp.where(d <= lmin, cols, jnp.int32(2 ** 30)),
                   axis=1, keepdims=True)
    larg = lidx + j * _CB

    @pl.when(j == 0)
    def _():
        val_ref[...] = lmin
        idx_ref[...] = larg

    @pl.when((j > 0) & (j % _SCHUNK == 0))
    def _():
        val_ref[...] = val_ref[...].astype(jnp.bfloat16).astype(jnp.float32)

    @pl.when(j > 0)
    def _():
        prev_v = val_ref[...]
        prev_i = idx_ref[...]
        better = lmin < prev_v
        val_ref[...] = jnp.where(better, lmin, prev_v)
        idx_ref[...] = jnp.where(better, larg, prev_i)


def _vq_argmin(xt, cb):
    T = xt.shape[0]
    n_tb = T // _TB
    n_cb = CODES_CNT // _CB
    vals, idxs = pl.pallas_call(
        _vq_dist_kernel,
        grid=(n_tb, n_cb),
        in_specs=[
            pl.BlockSpec((_TB, CODES_DIM), lambda i, j: (i, 0)),
            pl.BlockSpec((_CB, CODES_DIM), lambda i, j: (j, 0)),
        ],
        out_specs=[
            pl.BlockSpec((_TB, 1), lambda i, j: (i, 0)),
            pl.BlockSpec((_TB, 1), lambda i, j: (i, 0)),
        ],
        out_shape=[
            jax.ShapeDtypeStruct((T, 1), jnp.float32),
            jax.ShapeDtypeStruct((T, 1), jnp.int32),
        ],
    )(xt, cb)
    return vals.reshape(T), idxs.reshape(T)


def _sc_gather(cb, ind):
    info = plsc.get_sparse_core_info()
    ncores = info.num_cores
    nw = ncores * info.num_subcores
    B = ind.shape[0]
    D = cb.shape[1]
    bpw = B // nw
    mesh = plsc.VectorSubcoreMesh(core_axis_name="c", subcore_axis_name="s")

    @functools.partial(
        pl.kernel, mesh=mesh,
        out_type=jax.ShapeDtypeStruct((B, D), jnp.float32),
        scratch_types=[
            pltpu.VMEM((bpw,), jnp.int32),
            pltpu.VMEM((bpw, D), jnp.float32),
            pltpu.SemaphoreType.DMA,
        ],
    )
    def k(table_hbm, idx_hbm, out_hbm, idx_v, rows_v, sem):
        wid = jax.lax.axis_index("s") * ncores + jax.lax.axis_index("c")
        base = wid * bpw
        pltpu.sync_copy(idx_hbm.at[pl.ds(base, bpw)], idx_v)
        pltpu.async_copy(table_hbm.at[idx_v], rows_v, sem).wait()
        pltpu.sync_copy(rows_v, out_hbm.at[pl.ds(base, bpw)])

    return k(cb, ind)


def _conv1d(x, w, s, p):
    return jax.lax.conv_general_dilated(
        x, w, (s,), [(p, p)], dimension_numbers=('NCH', 'OIH', 'NCH'))


def _convT1d(x, w, s, p):
    k = w.shape[2]
    w2 = jnp.transpose(jnp.flip(w, 2), (1, 0, 2))
    return jax.lax.conv_general_dilated(
        x, w2, (1,), [(k - 1 - p, k - 1 - p)], lhs_dilation=(s,),
        dimension_numbers=('NCH', 'OIH', 'NCH'))


def _bn(x):
    m = jnp.mean(x, axis=(0, 2), keepdims=True)
    v = jnp.var(x, axis=(0, 2), keepdims=True)
    return (x - m) / jnp.sqrt(v + 1e-5)


def _resblock(x, w1, w2):
    h = jax.nn.relu(x)
    h = _conv1d(h, w1, 1, 1)
    h = _bn(h)
    h = jax.nn.relu(h)
    h = _conv1d(h, w2, 1, 0)
    return _bn(x + h)


def kernel(x, cb, ew1, ew2, er1a, er1b, er2a, er2b, ew3, ew4,
           dw1, dw2, dr1a, dr1b, dr2a, dr2b, dw3, dw4):
    h = _conv1d(x, ew1, 2, 1)
    h = jax.nn.relu(h)
    h = _conv1d(h, ew2, 2, 1)
    h = _bn(h)
    h = _resblock(h, er1a, er1b)
    h = _resblock(h, er2a, er2b)
    h = jax.nn.relu(h)
    h = _conv1d(h, ew3, 2, 1)
    h = _bn(h)
    h = jax.nn.relu(h)
    z = _conv1d(h, ew4, 1, 1)

    b, c, L = z.shape
    xt = jnp.transpose(z, (1, 0, 2)).reshape(c, -1).T

    mind, ind = _vq_argmin(xt, cb)
    loss = (1.0 + COMMIT) * jnp.sum(mind) / (c * b * L)

    cb_pad = jnp.concatenate([cb, jnp.zeros_like(cb)], axis=1)
    qs = _sc_gather(cb_pad, ind)[:, :CODES_DIM]
    zq = jnp.transpose(qs.T.reshape(c, b, L), (1, 0, 2))

    h = _convT1d(zq, dw1, 1, 1)
    h = _bn(h)
    h = jax.nn.relu(h)
    h = _convT1d(h, dw2, 2, 1)
    h = _bn(h)
    h = _resblock(h, dr1a, dr1b)
    h = _resblock(h, dr2a, dr2b)
    h = jax.nn.relu(h)
    h = _convT1d(h, dw3, 2, 1)
    h = _bn(h)
    h = jax.nn.relu(h)
    h = _convT1d(h, dw4, 2, 1)
    x_rec = jnp.tanh(h)
    return (loss, x_rec)


# speedup vs baseline: 2.1341x; 1.0420x over previous
import functools

import jax
import jax.numpy as jnp
from jax.experimental import pallas as pl
from jax.experimental.pallas import tpu as pltpu
from jax.experimental.pallas import tpu_sc as plsc

CODES_DIM = 64
CODES_CNT = 8192
COMMIT = 0.25


_TB = 2048
_CB = 2048
_SCHUNK = 2048 // _CB


def _vq_dist_kernel(x_ref, cb_ref, val_ref, idx_ref):
    j = pl.program_id(1)
    x = x_ref[...]
    cbk = cb_ref[...]
    xy = jax.lax.dot_general(x.astype(jnp.bfloat16), cbk.astype(jnp.bfloat16),
                             (((1,), (1,)), ((), ())),
                             preferred_element_type=jnp.float32)
    x2 = jnp.sum(x * x, axis=1)[:, None]
    y2 = jnp.sum(cbk * cbk, axis=1)[None, :]
    d = (x2 + y2) - 2.0 * xy
    lmin = jnp.min(d, axis=1, keepdims=True)
    cols = jax.lax.broadcasted_iota(jnp.int32, (_TB, _CB), 1).astype(jnp.float32)
    lidx = jnp.min(jnp.where(d <= lmin, cols, jnp.float32(2.0 ** 30)),
                   axis=1, keepdims=True)
    larg = lidx.astype(jnp.int32) + j * _CB

    @pl.when(j == 0)
    def _():
        val_ref[...] = lmin
        idx_ref[...] = larg

    @pl.when((j > 0) & (j % _SCHUNK == 0))
    def _():
        val_ref[...] = val_ref[...].astype(jnp.bfloat16).astype(jnp.float32)

    @pl.when(j > 0)
    def _():
        prev_v = val_ref[...]
        prev_i = idx_ref[...]
        better = lmin < prev_v
        val_ref[...] = jnp.where(better, lmin, prev_v)
        idx_ref[...] = jnp.where(better, larg, prev_i)


def _vq_argmin(xt, cb):
    T = xt.shape[0]
    n_tb = T // _TB
    n_cb = CODES_CNT // _CB
    vals, idxs = pl.pallas_call(
        _vq_dist_kernel,
        grid=(n_tb, n_cb),
        in_specs=[
            pl.BlockSpec((_TB, CODES_DIM), lambda i, j: (i, 0)),
            pl.BlockSpec((_CB, CODES_DIM), lambda i, j: (j, 0)),
        ],
        out_specs=[
            pl.BlockSpec((_TB, 1), lambda i, j: (i, 0)),
            pl.BlockSpec((_TB, 1), lambda i, j: (i, 0)),
        ],
        out_shape=[
            jax.ShapeDtypeStruct((T, 1), jnp.float32),
            jax.ShapeDtypeStruct((T, 1), jnp.int32),
        ],
    )(xt, cb)
    return vals.reshape(T), idxs.reshape(T)


def _sc_gather(cb, ind):
    info = plsc.get_sparse_core_info()
    ncores = info.num_cores
    nw = ncores * info.num_subcores
    B = ind.shape[0]
    D = cb.shape[1]
    bpw = B // nw
    mesh = plsc.VectorSubcoreMesh(core_axis_name="c", subcore_axis_name="s")

    @functools.partial(
        pl.kernel, mesh=mesh,
        out_type=jax.ShapeDtypeStruct((B, D), jnp.float32),
        scratch_types=[
            pltpu.VMEM((bpw,), jnp.int32),
            pltpu.VMEM((bpw, D), jnp.float32),
            pltpu.SemaphoreType.DMA,
        ],
    )
    def k(table_hbm, idx_hbm, out_hbm, idx_v, rows_v, sem):
        wid = jax.lax.axis_index("s") * ncores + jax.lax.axis_index("c")
        base = wid * bpw
        pltpu.sync_copy(idx_hbm.at[pl.ds(base, bpw)], idx_v)
        pltpu.async_copy(table_hbm.at[idx_v], rows_v, sem).wait()
        pltpu.sync_copy(rows_v, out_hbm.at[pl.ds(base, bpw)])

    return k(cb, ind)


def _conv1d(x, w, s, p):
    return jax.lax.conv_general_dilated(
        x, w, (s,), [(p, p)], dimension_numbers=('NCH', 'OIH', 'NCH'))


def _convT1d(x, w, s, p):
    k = w.shape[2]
    w2 = jnp.transpose(jnp.flip(w, 2), (1, 0, 2))
    return jax.lax.conv_general_dilated(
        x, w2, (1,), [(k - 1 - p, k - 1 - p)], lhs_dilation=(s,),
        dimension_numbers=('NCH', 'OIH', 'NCH'))


def _bn(x):
    m = jnp.mean(x, axis=(0, 2), keepdims=True)
    v = jnp.var(x, axis=(0, 2), keepdims=True)
    return (x - m) / jnp.sqrt(v + 1e-5)


def _resblock(x, w1, w2):
    h = jax.nn.relu(x)
    h = _conv1d(h, w1, 1, 1)
    h = _bn(h)
    h = jax.nn.relu(h)
    h = _conv1d(h, w2, 1, 0)
    return _bn(x + h)


def kernel(x, cb, ew1, ew2, er1a, er1b, er2a, er2b, ew3, ew4,
           dw1, dw2, dr1a, dr1b, dr2a, dr2b, dw3, dw4):
    h = _conv1d(x, ew1, 2, 1)
    h = jax.nn.relu(h)
    h = _conv1d(h, ew2, 2, 1)
    h = _bn(h)
    h = _resblock(h, er1a, er1b)
    h = _resblock(h, er2a, er2b)
    h = jax.nn.relu(h)
    h = _conv1d(h, ew3, 2, 1)
    h = _bn(h)
    h = jax.nn.relu(h)
    z = _conv1d(h, ew4, 1, 1)

    b, c, L = z.shape
    xt = jnp.transpose(z, (1, 0, 2)).reshape(c, -1).T

    mind, ind = _vq_argmin(xt, cb)
    loss = (1.0 + COMMIT) * jnp.sum(mind) / (c * b * L)

    cb_pad = jnp.concatenate([cb, jnp.zeros_like(cb)], axis=1)
    qs = _sc_gather(cb_pad, ind)[:, :CODES_DIM]
    zq = jnp.transpose(qs.T.reshape(c, b, L), (1, 0, 2))

    h = _convT1d(zq, dw1, 1, 1)
    h = _bn(h)
    h = jax.nn.relu(h)
    h = _convT1d(h, dw2, 2, 1)
    h = _bn(h)
    h = _resblock(h, dr1a, dr1b)
    h = _resblock(h, dr2a, dr2b)
    h = jax.nn.relu(h)
    h = _convT1d(h, dw3, 2, 1)
    h = _bn(h)
    h = jax.nn.relu(h)
    h = _convT1d(h, dw4, 2, 1)
    x_rec = jnp.tanh(h)
    return (loss, x_rec)
